# baseline (speedup 1.0000x reference)
import functools

import jax
import jax.numpy as jnp
from jax import lax
from jax.experimental import pallas as pl
from jax.experimental.pallas import tpu as pltpu
from jax.experimental.pallas import tpu_sc as plsc

NC = 2
NS = 16
LANES = 16


def _zero_fill(ref, rows, cols):
    zv = jnp.zeros((LANES,), jnp.float32)
    cpr = cols // LANES

    def body(i, c):
        r = i // cpr
        col = (i % cpr) * LANES
        ref[r, pl.ds(col, LANES)] = zv
        return c

    lax.fori_loop(0, rows * cpr, body, 0)


def _make_edge_agg(n_nodes, n_edges, n_ch):
    nw = NC * NS
    epw = n_edges // nw
    K = 80
    assert n_edges % nw == 0 and epw % K == 0
    nchunks = epw // K
    stripe = n_nodes // NS
    assert n_nodes % NS == 0 and stripe % 8 == 0
    zr = 128
    assert stripe % zr == 0

    mesh = plsc.VectorSubcoreMesh(core_axis_name="c", subcore_axis_name="s")

    nbuf = 3
    scratch = [
        [pltpu.VMEM((K,), jnp.int32) for _ in range(nbuf)],
        [pltpu.VMEM((K,), jnp.int32) for _ in range(nbuf)],
        [pltpu.VMEM((K, n_ch), jnp.float32) for _ in range(nbuf)],
        pltpu.VMEM((zr, n_ch), jnp.float32),
        pltpu.VMEM_SHARED((n_nodes, n_ch), jnp.float32),
        [pltpu.SemaphoreType.DMA for _ in range(nbuf)],
        [pltpu.SemaphoreType.DMA for _ in range(nbuf)],
    ]

    def body(table_hbm, src_hbm, dst_hbm, out_hbm,
             src_v, dst_v, rows_v, z_v, acc_sh, gsem, ssem):
        cid = lax.axis_index("c")
        sid = lax.axis_index("s")
        wid = sid * NC + cid

        _zero_fill(z_v, zr, n_ch)
        def zcopy(j, c):
            pltpu.sync_copy(z_v, acc_sh.at[pl.ds(sid * stripe + j * zr, zr)])
            return c
        lax.fori_loop(0, stripe // zr, zcopy, 0)

        plsc.subcore_barrier()

        base = wid * epw

        def start(g, b):
            off = pl.multiple_of(base + g * K, 8)
            pltpu.sync_copy(dst_hbm.at[pl.ds(off, K)], dst_v[b])
            pltpu.sync_copy(src_hbm.at[pl.ds(off, K)], src_v[b])
            pltpu.async_copy(table_hbm.at[src_v[b]], rows_v[b], gsem[b])

        def finish(b):
            pltpu.make_async_copy(table_hbm.at[src_v[b]], rows_v[b],
                                  gsem[b]).wait()
            pltpu.async_copy(rows_v[b], acc_sh.at[dst_v[b]], ssem[b],
                             add=True)

        def wait_scatter(b):
            pltpu.make_async_copy(rows_v[b], acc_sh.at[dst_v[b]],
                                  ssem[b]).wait()

        la = nbuf - 1
        for g in range(la):
            start(g, g)

        start(la, la % nbuf)
        finish(0)

        def outer(g0, c):
            for db in range(nbuf):
                g = g0 + db
                b = (1 + db) % nbuf

                @pl.when(g < nchunks)
                def _():
                    @pl.when(g + la < nchunks)
                    def _():
                        nb = (b + la) % nbuf
                        wait_scatter(nb)
                        start(g + la, nb)
                    finish(b)
            return c
        lax.fori_loop(0, (nchunks - 1 + nbuf - 1) // nbuf,
                      lambda i, c: outer(1 + i * nbuf, c), 0)

        for b in range(nbuf):
            wait_scatter(b)

        plsc.subcore_barrier()

        row0 = sid * stripe
        pltpu.sync_copy(acc_sh.at[pl.ds(row0, stripe)],
                        out_hbm.at[cid, pl.ds(row0, stripe)])

    return pl.kernel(
        body,
        out_type=jax.ShapeDtypeStruct((NC, n_nodes, n_ch), jnp.float32),
        mesh=mesh,
        scratch_types=scratch,
    )


def _make_deg_hist(n_nodes, n_edges):
    nw = NC * NS
    epw = n_edges // nw
    K = 80
    assert n_edges % nw == 0 and epw % K == 0
    nchunks = epw // K
    stripe = n_nodes // NS
    assert n_nodes % NS == 0 and stripe % 8 == 0

    mesh = plsc.VectorSubcoreMesh(core_axis_name="c", subcore_axis_name="s")

    nbuf = 3

    def body(dst_hbm, out_hbm, dst_v, ones_v, z_v, acc_sh, sem):
        cid = lax.axis_index("c")
        sid = lax.axis_index("s")
        wid = sid * NC + cid
        zv = jnp.zeros((LANES,), jnp.float32)
        ov = jnp.ones((LANES,), jnp.float32)

        def zf(i, c):
            z_v[pl.ds(i * LANES, LANES)] = zv
            return c
        lax.fori_loop(0, stripe // LANES, zf, 0)

        def of(i, c):
            ones_v[pl.ds(i * LANES, LANES)] = ov
            return c
        lax.fori_loop(0, K // LANES, of, 0)

        pltpu.sync_copy(z_v, acc_sh.at[pl.ds(sid * stripe, stripe)])
        plsc.subcore_barrier()

        base = wid * epw

        def start(g, b):
            off = pl.multiple_of(base + g * K, 8)
            pltpu.async_copy(dst_hbm.at[pl.ds(off, K)], dst_v[b], sem[b])

        def finish(g, b):
            off = pl.multiple_of(base + g * K, 8)
            pltpu.make_async_copy(dst_hbm.at[pl.ds(off, K)], dst_v[b],
                                  sem[b]).wait()
            pltpu.sync_copy(ones_v, acc_sh.at[dst_v[b]], add=True)

        la = nbuf - 1
        for g in range(la):
            start(g, g)

        def outer(g0, c):
            for b in range(nbuf):
                g = g0 + b

                @pl.when(g < nchunks)
                def _():
                    @pl.when(g + la < nchunks)
                    def _():
                        start(g + la, (b + la) % nbuf)
                    finish(g, b)
            return c
        lax.fori_loop(0, (nchunks + nbuf - 1) // nbuf,
                      lambda i, c: outer(i * nbuf, c), 0)

        plsc.subcore_barrier()
        pltpu.sync_copy(acc_sh.at[pl.ds(sid * stripe, stripe)],
                        out_hbm.at[cid, pl.ds(sid * stripe, stripe)])

    return pl.kernel(
        body,
        out_type=jax.ShapeDtypeStruct((NC, n_nodes), jnp.float32),
        mesh=mesh,
        scratch_types=[
            [pltpu.VMEM((K,), jnp.int32) for _ in range(nbuf)],
            pltpu.VMEM((K,), jnp.float32),
            pltpu.VMEM((stripe,), jnp.float32),
            pltpu.VMEM_SHARED((n_nodes,), jnp.float32),
            [pltpu.SemaphoreType.DMA for _ in range(nbuf)],
        ],
    )


def _scale_kernel(d0_ref, d1_ref, x_ref, dis_ref, xs_ref):
    deg = d0_ref[...] + d1_ref[...] + 1.0
    dis = lax.rsqrt(deg)
    dis_ref[...] = dis
    xs_ref[...] = x_ref[...] * dis


def _pre_kernel(xs_ref, dis_ref, w1_ref, b_ref):
    b_ref[...] = jnp.dot(dis_ref[...] * xs_ref[...], w1_ref[...],
                         preferred_element_type=jnp.float32)


def _mlp_kernel(p0_ref, p1_ref, bb_ref, dis_ref, w1_ref, b1_ref, w2_ref,
                ts_ref):
    dis = dis_ref[...]
    a = dis * (p0_ref[...] + p1_ref[...])
    h = jnp.maximum(
        jnp.dot(a, w1_ref[...], preferred_element_type=jnp.float32)
        + bb_ref[...] + b1_ref[...], 0.0)
    t = jnp.dot(h, w2_ref[...], preferred_element_type=jnp.float32)
    br, oc = t.shape
    ts_ref[...] = jnp.concatenate(
        [dis * t, jnp.zeros((br, 128 - oc), jnp.float32)], axis=1)


def _out_kernel(q0_ref, q1_ref, ts_ref, dis_ref, b2_ref, o_ref):
    oc = o_ref.shape[1]
    v = (dis_ref[...] * (q0_ref[:, :oc] + q1_ref[:, :oc] + ts_ref[:, :oc])
         + b2_ref[...])
    m = jnp.max(v, axis=1, keepdims=True)
    e = jnp.exp(v - m)
    s = jnp.sum(e, axis=1, keepdims=True)
    o_ref[...] = v - m - jnp.log(s)


def _rows(br, c):
    return pl.BlockSpec((br, c), lambda i: (i, 0))


def _full(shape):
    return pl.BlockSpec(shape, lambda i: tuple(0 for _ in shape))


def kernel(x, edge_index, W1, b1, W2, b2):
    n, in_ch = x.shape
    hid = W1.shape[1]
    out_ch = W2.shape[1]
    e = edge_index.shape[1]
    src = edge_index[0].astype(jnp.int32)
    dst = edge_index[1].astype(jnp.int32)

    align = NS * 128
    npad = ((n + align - 1) // align) * align
    xp = jnp.pad(x, ((0, npad - n), (0, 0)))

    br = 1024
    grid = (npad // br,)

    degp = _make_deg_hist(npad, e)(dst)

    dis, xs = pl.pallas_call(
        _scale_kernel,
        grid=grid,
        in_specs=[_rows(br, 1), _rows(br, 1), _rows(br, in_ch)],
        out_specs=[_rows(br, 1), _rows(br, in_ch)],
        out_shape=[jax.ShapeDtypeStruct((npad, 1), jnp.float32),
                   jax.ShapeDtypeStruct((npad, in_ch), jnp.float32)],
    )(degp[0].reshape(npad, 1), degp[1].reshape(npad, 1), xp)

    bb = pl.pallas_call(
        _pre_kernel,
        grid=grid,
        in_specs=[_rows(br, in_ch), _rows(br, 1), _full((in_ch, hid))],
        out_specs=_rows(br, hid),
        out_shape=jax.ShapeDtypeStruct((npad, hid), jnp.float32),
    )(xs, dis, W1)

    p = _make_edge_agg(npad, e, in_ch)(xs, src, dst)

    ts = pl.pallas_call(
        _mlp_kernel,
        grid=grid,
        in_specs=[_rows(br, in_ch), _rows(br, in_ch), _rows(br, hid),
                  _rows(br, 1), _full((in_ch, hid)), _full((1, hid)),
                  _full((hid, out_ch))],
        out_specs=_rows(br, 128),
        out_shape=jax.ShapeDtypeStruct((npad, 128), jnp.float32),
    )(p[0], p[1], bb, dis, W1, b1.reshape(1, hid), W2)

    q = _make_edge_agg(npad, e, 128)(ts, src, dst)

    out = pl.pallas_call(
        _out_kernel,
        grid=grid,
        in_specs=[_rows(br, 128), _rows(br, 128), _rows(br, 128),
                  _rows(br, 1), _full((1, out_ch))],
        out_specs=_rows(br, out_ch),
        out_shape=jax.ShapeDtypeStruct((npad, out_ch), jnp.float32),
    )(q[0], q[1], ts, dis, b2.reshape(1, out_ch))

    return out[:n]


# speedup vs baseline: 1.0050x; 1.0050x over previous
import functools

import jax
import jax.numpy as jnp
from jax import lax
from jax.experimental import pallas as pl
from jax.experimental.pallas import tpu as pltpu
from jax.experimental.pallas import tpu_sc as plsc

NC = 2
NS = 16
LANES = 16


def _zero_fill(ref, rows, cols):
    zv = jnp.zeros((LANES,), jnp.float32)
    cpr = cols // LANES

    def body(i, c):
        r = i // cpr
        col = (i % cpr) * LANES
        ref[r, pl.ds(col, LANES)] = zv
        return c

    lax.fori_loop(0, rows * cpr, body, 0)


def _make_edge_agg(n_nodes, n_edges, n_ch):
    nw = NC * NS
    epw = n_edges // nw
    K = 80
    assert n_edges % nw == 0 and epw % K == 0
    nchunks = epw // K
    stripe = n_nodes // NS
    assert n_nodes % NS == 0 and stripe % 8 == 0
    zr = 128
    assert stripe % zr == 0

    mesh = plsc.VectorSubcoreMesh(core_axis_name="c", subcore_axis_name="s")

    nbuf = 3
    scratch = [
        [pltpu.VMEM((K,), jnp.int32) for _ in range(nbuf)],
        [pltpu.VMEM((K,), jnp.int32) for _ in range(nbuf)],
        [pltpu.VMEM((K, n_ch), jnp.float32) for _ in range(nbuf)],
        pltpu.VMEM((zr, n_ch), jnp.float32),
        pltpu.VMEM_SHARED((n_nodes, n_ch), jnp.float32),
        [pltpu.SemaphoreType.DMA for _ in range(nbuf)],
        [pltpu.SemaphoreType.DMA for _ in range(nbuf)],
    ]

    def body(table_hbm, src_hbm, dst_hbm, out_hbm,
             src_v, dst_v, rows_v, z_v, acc_sh, gsem, ssem):
        cid = lax.axis_index("c")
        sid = lax.axis_index("s")
        wid = sid * NC + cid

        _zero_fill(z_v, zr, n_ch)
        def zcopy(j, c):
            pltpu.sync_copy(z_v, acc_sh.at[pl.ds(sid * stripe + j * zr, zr)])
            return c
        lax.fori_loop(0, stripe // zr, zcopy, 0)

        plsc.subcore_barrier()

        base = wid * epw

        def start(g, b):
            off = pl.multiple_of(base + g * K, 8)
            pltpu.sync_copy(dst_hbm.at[pl.ds(off, K)], dst_v[b])
            pltpu.sync_copy(src_hbm.at[pl.ds(off, K)], src_v[b])
            pltpu.async_copy(table_hbm.at[src_v[b]], rows_v[b], gsem[b])

        def finish(b):
            pltpu.make_async_copy(table_hbm.at[src_v[b]], rows_v[b],
                                  gsem[b]).wait()
            pltpu.async_copy(rows_v[b], acc_sh.at[dst_v[b]], ssem[b],
                             add=True)

        def wait_scatter(b):
            pltpu.make_async_copy(rows_v[b], acc_sh.at[dst_v[b]],
                                  ssem[b]).wait()

        la = nbuf - 1
        for g in range(la):
            start(g, g)

        start(la, la % nbuf)
        finish(0)

        def outer(g0, c):
            for db in range(nbuf):
                g = g0 + db
                b = (1 + db) % nbuf

                @pl.when(g < nchunks)
                def _():
                    @pl.when(g + la < nchunks)
                    def _():
                        nb = (b + la) % nbuf
                        wait_scatter(nb)
                        start(g + la, nb)
                    finish(b)
            return c
        lax.fori_loop(0, (nchunks - 1 + nbuf - 1) // nbuf,
                      lambda i, c: outer(1 + i * nbuf, c), 0)

        for b in range(nbuf):
            wait_scatter(b)

        plsc.subcore_barrier()

        row0 = sid * stripe
        pltpu.sync_copy(acc_sh.at[pl.ds(row0, stripe)],
                        out_hbm.at[cid, pl.ds(row0, stripe)])

    return pl.kernel(
        body,
        out_type=jax.ShapeDtypeStruct((NC, n_nodes, n_ch), jnp.float32),
        mesh=mesh,
        scratch_types=scratch,
    )


def _make_deg_hist(n_nodes, n_edges):
    nw = NC * NS
    epw = n_edges // nw
    K = 80
    assert n_edges % nw == 0 and epw % K == 0
    nchunks = epw // K
    stripe = n_nodes // NS
    assert n_nodes % NS == 0 and stripe % 8 == 0

    mesh = plsc.VectorSubcoreMesh(core_axis_name="c", subcore_axis_name="s")

    nbuf = 3

    def body(dst_hbm, out_hbm, dst_v, ones_v, z_v, acc_sh, sem):
        cid = lax.axis_index("c")
        sid = lax.axis_index("s")
        wid = sid * NC + cid
        zv = jnp.zeros((LANES,), jnp.float32)
        ov = jnp.ones((LANES,), jnp.float32)

        def zf(i, c):
            z_v[pl.ds(i * LANES, LANES)] = zv
            return c
        lax.fori_loop(0, stripe // LANES, zf, 0)

        def of(i, c):
            ones_v[pl.ds(i * LANES, LANES)] = ov
            return c
        lax.fori_loop(0, K // LANES, of, 0)

        pltpu.sync_copy(z_v, acc_sh.at[pl.ds(sid * stripe, stripe)])
        plsc.subcore_barrier()

        base = wid * epw

        def start(g, b):
            off = pl.multiple_of(base + g * K, 8)
            pltpu.async_copy(dst_hbm.at[pl.ds(off, K)], dst_v[b], sem[b])

        def finish(g, b):
            off = pl.multiple_of(base + g * K, 8)
            pltpu.make_async_copy(dst_hbm.at[pl.ds(off, K)], dst_v[b],
                                  sem[b]).wait()
            pltpu.sync_copy(ones_v, acc_sh.at[dst_v[b]], add=True)

        la = nbuf - 1
        for g in range(la):
            start(g, g)

        def outer(g0, c):
            for b in range(nbuf):
                g = g0 + b

                @pl.when(g < nchunks)
                def _():
                    @pl.when(g + la < nchunks)
                    def _():
                        start(g + la, (b + la) % nbuf)
                    finish(g, b)
            return c
        lax.fori_loop(0, (nchunks + nbuf - 1) // nbuf,
                      lambda i, c: outer(i * nbuf, c), 0)

        plsc.subcore_barrier()
        pltpu.sync_copy(acc_sh.at[pl.ds(sid * stripe, stripe)],
                        out_hbm.at[cid, pl.ds(sid * stripe, stripe)])

    return pl.kernel(
        body,
        out_type=jax.ShapeDtypeStruct((NC, n_nodes), jnp.float32),
        mesh=mesh,
        scratch_types=[
            [pltpu.VMEM((K,), jnp.int32) for _ in range(nbuf)],
            pltpu.VMEM((K,), jnp.float32),
            pltpu.VMEM((stripe,), jnp.float32),
            pltpu.VMEM_SHARED((n_nodes,), jnp.float32),
            [pltpu.SemaphoreType.DMA for _ in range(nbuf)],
        ],
    )


def _scale_kernel(d0_ref, d1_ref, x_ref, dis_ref, xs_ref):
    deg = d0_ref[...] + d1_ref[...] + 1.0
    dis = lax.rsqrt(deg)
    dis_ref[...] = dis
    xs_ref[...] = x_ref[...] * dis


def _mlp_kernel(p0_ref, p1_ref, xs_ref, dis_ref, w1_ref, b1_ref, w2_ref,
                ts_ref):
    dis = dis_ref[...]
    a = dis * (p0_ref[...] + p1_ref[...] + xs_ref[...])
    h = jnp.maximum(
        jnp.dot(a, w1_ref[...], preferred_element_type=jnp.float32)
        + b1_ref[...], 0.0)
    t = jnp.dot(h, w2_ref[...], preferred_element_type=jnp.float32)
    br, oc = t.shape
    ts_ref[...] = jnp.concatenate(
        [dis * t, jnp.zeros((br, 128 - oc), jnp.float32)], axis=1)


def _out_kernel(q0_ref, q1_ref, ts_ref, dis_ref, b2_ref, o_ref):
    oc = o_ref.shape[1]
    v = (dis_ref[...] * (q0_ref[:, :oc] + q1_ref[:, :oc] + ts_ref[:, :oc])
         + b2_ref[...])
    m = jnp.max(v, axis=1, keepdims=True)
    e = jnp.exp(v - m)
    s = jnp.sum(e, axis=1, keepdims=True)
    o_ref[...] = v - m - jnp.log(s)


def _rows(br, c):
    return pl.BlockSpec((br, c), lambda i: (i, 0))


def _full(shape):
    return pl.BlockSpec(shape, lambda i: tuple(0 for _ in shape))


def kernel(x, edge_index, W1, b1, W2, b2):
    n, in_ch = x.shape
    hid = W1.shape[1]
    out_ch = W2.shape[1]
    e = edge_index.shape[1]
    src = edge_index[0].astype(jnp.int32)
    dst = edge_index[1].astype(jnp.int32)

    align = NS * 128
    npad = ((n + align - 1) // align) * align
    xp = jnp.pad(x, ((0, npad - n), (0, 0)))

    br = 1024
    grid = (npad // br,)

    degp = _make_deg_hist(npad, e)(dst)

    dis, xs = pl.pallas_call(
        _scale_kernel,
        grid=grid,
        in_specs=[_rows(br, 1), _rows(br, 1), _rows(br, in_ch)],
        out_specs=[_rows(br, 1), _rows(br, in_ch)],
        out_shape=[jax.ShapeDtypeStruct((npad, 1), jnp.float32),
                   jax.ShapeDtypeStruct((npad, in_ch), jnp.float32)],
    )(degp[0].reshape(npad, 1), degp[1].reshape(npad, 1), xp)

    p = _make_edge_agg(npad, e, in_ch)(xs, src, dst)

    ts = pl.pallas_call(
        _mlp_kernel,
        grid=grid,
        in_specs=[_rows(br, in_ch), _rows(br, in_ch), _rows(br, in_ch),
                  _rows(br, 1), _full((in_ch, hid)), _full((1, hid)),
                  _full((hid, out_ch))],
        out_specs=_rows(br, 128),
        out_shape=jax.ShapeDtypeStruct((npad, 128), jnp.float32),
    )(p[0], p[1], xs, dis, W1, b1.reshape(1, hid), W2)

    q = _make_edge_agg(npad, e, 128)(ts, src, dst)

    out = pl.pallas_call(
        _out_kernel,
        grid=grid,
        in_specs=[_rows(br, 128), _rows(br, 128), _rows(br, 128),
                  _rows(br, 1), _full((1, out_ch))],
        out_specs=_rows(br, out_ch),
        out_shape=jax.ShapeDtypeStruct((npad, out_ch), jnp.float32),
    )(q[0], q[1], ts, dis, b2.reshape(1, out_ch))

    return out[:n]


# speedup vs baseline: 1.0536x; 1.0484x over previous
import functools

import jax
import jax.numpy as jnp
from jax import lax
from jax.experimental import pallas as pl
from jax.experimental.pallas import tpu as pltpu
from jax.experimental.pallas import tpu_sc as plsc

NC = 2
NS = 16
LANES = 16


def _zero_fill(ref, rows, cols):
    zv = jnp.zeros((LANES,), jnp.float32)
    cpr = cols // LANES

    def body(i, c):
        r = i // cpr
        col = (i % cpr) * LANES
        ref[r, pl.ds(col, LANES)] = zv
        return c

    lax.fori_loop(0, rows * cpr, body, 0)


def _make_edge_agg(n_nodes, n_edges, n_ch):
    nw = NC * NS
    epw = n_edges // nw
    K = 80
    assert n_edges % nw == 0 and epw % K == 0
    nchunks = epw // K
    stripe = n_nodes // NS
    assert n_nodes % NS == 0 and stripe % 8 == 0
    zr = 128
    assert stripe % zr == 0

    mesh = plsc.VectorSubcoreMesh(core_axis_name="c", subcore_axis_name="s")

    nbuf = 3
    scratch = [
        [pltpu.VMEM((K,), jnp.int32) for _ in range(nbuf)],
        [pltpu.VMEM((K,), jnp.int32) for _ in range(nbuf)],
        [pltpu.VMEM((K, n_ch), jnp.float32) for _ in range(nbuf)],
        pltpu.VMEM((zr, n_ch), jnp.float32),
        pltpu.VMEM_SHARED((n_nodes, n_ch), jnp.float32),
        [pltpu.SemaphoreType.DMA for _ in range(nbuf)],
        [pltpu.SemaphoreType.DMA for _ in range(nbuf)],
    ]

    def body(table_hbm, src_hbm, dst_hbm, out_hbm,
             src_v, dst_v, rows_v, z_v, acc_sh, gsem, ssem):
        cid = lax.axis_index("c")
        sid = lax.axis_index("s")
        wid = sid * NC + cid

        _zero_fill(z_v, zr, n_ch)
        def zcopy(j, c):
            pltpu.sync_copy(z_v, acc_sh.at[pl.ds(sid * stripe + j * zr, zr)])
            return c
        lax.fori_loop(0, stripe // zr, zcopy, 0)

        plsc.subcore_barrier()

        base = wid * epw

        def start(g, b):
            off = pl.multiple_of(base + g * K, 8)
            pltpu.sync_copy(dst_hbm.at[pl.ds(off, K)], dst_v[b])
            pltpu.sync_copy(src_hbm.at[pl.ds(off, K)], src_v[b])
            pltpu.async_copy(table_hbm.at[src_v[b]], rows_v[b], gsem[b])

        def finish(b):
            pltpu.make_async_copy(table_hbm.at[src_v[b]], rows_v[b],
                                  gsem[b]).wait()
            pltpu.async_copy(rows_v[b], acc_sh.at[dst_v[b]], ssem[b],
                             add=True)

        def wait_scatter(b):
            pltpu.make_async_copy(rows_v[b], acc_sh.at[dst_v[b]],
                                  ssem[b]).wait()

        la = nbuf - 1
        for g in range(la):
            start(g, g)

        start(la, la % nbuf)
        finish(0)

        def outer(g0, c):
            for db in range(nbuf):
                g = g0 + db
                b = (1 + db) % nbuf

                @pl.when(g < nchunks)
                def _():
                    @pl.when(g + la < nchunks)
                    def _():
                        nb = (b + la) % nbuf
                        wait_scatter(nb)
                        start(g + la, nb)
                    finish(b)
            return c
        lax.fori_loop(0, (nchunks - 1 + nbuf - 1) // nbuf,
                      lambda i, c: outer(1 + i * nbuf, c), 0)

        for b in range(nbuf):
            wait_scatter(b)

        plsc.subcore_barrier()

        row0 = sid * stripe
        pltpu.sync_copy(acc_sh.at[pl.ds(row0, stripe)],
                        out_hbm.at[cid, pl.ds(row0, stripe)])

    return pl.kernel(
        body,
        out_type=jax.ShapeDtypeStruct((NC, n_nodes, n_ch), jnp.float32),
        mesh=mesh,
        scratch_types=scratch,
    )


def _make_deg_hist(n_nodes, n_edges):
    nw = NC * NS
    epw = n_edges // nw
    K = 80
    assert n_edges % nw == 0 and epw % K == 0
    nchunks = epw // K
    stripe = n_nodes // NS
    assert n_nodes % NS == 0 and stripe % 8 == 0

    mesh = plsc.VectorSubcoreMesh(core_axis_name="c", subcore_axis_name="s")

    nbuf = 3

    def body(dst_hbm, out_hbm, dst_v, ones_v, z_v, acc_sh, sem):
        cid = lax.axis_index("c")
        sid = lax.axis_index("s")
        wid = sid * NC + cid
        zv = jnp.zeros((LANES,), jnp.float32)
        ov = jnp.ones((LANES,), jnp.float32)

        def zf(i, c):
            z_v[pl.ds(i * LANES, LANES)] = zv
            return c
        lax.fori_loop(0, stripe // LANES, zf, 0)

        def of(i, c):
            ones_v[pl.ds(i * LANES, LANES)] = ov
            return c
        lax.fori_loop(0, K // LANES, of, 0)

        pltpu.sync_copy(z_v, acc_sh.at[pl.ds(sid * stripe, stripe)])
        plsc.subcore_barrier()

        base = wid * epw

        def start(g, b):
            off = pl.multiple_of(base + g * K, 8)
            pltpu.async_copy(dst_hbm.at[pl.ds(off, K)], dst_v[b], sem[b])

        def finish(g, b):
            off = pl.multiple_of(base + g * K, 8)
            pltpu.make_async_copy(dst_hbm.at[pl.ds(off, K)], dst_v[b],
                                  sem[b]).wait()
            pltpu.sync_copy(ones_v, acc_sh.at[dst_v[b]], add=True)

        la = nbuf - 1
        for g in range(la):
            start(g, g)

        def outer(g0, c):
            for b in range(nbuf):
                g = g0 + b

                @pl.when(g < nchunks)
                def _():
                    @pl.when(g + la < nchunks)
                    def _():
                        start(g + la, (b + la) % nbuf)
                    finish(g, b)
            return c
        lax.fori_loop(0, (nchunks + nbuf - 1) // nbuf,
                      lambda i, c: outer(i * nbuf, c), 0)

        plsc.subcore_barrier()
        pltpu.sync_copy(acc_sh.at[pl.ds(sid * stripe, stripe)],
                        out_hbm.at[cid, pl.ds(sid * stripe, stripe)])

    return pl.kernel(
        body,
        out_type=jax.ShapeDtypeStruct((NC, n_nodes), jnp.float32),
        mesh=mesh,
        scratch_types=[
            [pltpu.VMEM((K,), jnp.int32) for _ in range(nbuf)],
            pltpu.VMEM((K,), jnp.float32),
            pltpu.VMEM((stripe,), jnp.float32),
            pltpu.VMEM_SHARED((n_nodes,), jnp.float32),
            [pltpu.SemaphoreType.DMA for _ in range(nbuf)],
        ],
    )


def _scale_kernel(d0_ref, d1_ref, x_ref, dis_ref, xs_ref):
    deg = d0_ref[0] + d1_ref[0] + 1.0
    dis = lax.rsqrt(deg)
    dis_ref[...] = dis
    xs_ref[...] = x_ref[...] * dis


def _mlp_kernel(p0_ref, p1_ref, xs_ref, dis_ref, w1_ref, b1_ref, w2_ref,
                ts_ref):
    dis = dis_ref[...]
    a = dis * (p0_ref[0] + p1_ref[0] + xs_ref[...])
    h = jnp.maximum(
        jnp.dot(a, w1_ref[...], preferred_element_type=jnp.float32)
        + b1_ref[...], 0.0)
    t = jnp.dot(h, w2_ref[...], preferred_element_type=jnp.float32)
    br, oc = t.shape
    ts_ref[...] = jnp.concatenate(
        [dis * t, jnp.zeros((br, 128 - oc), jnp.float32)], axis=1)


def _out_kernel(q0_ref, q1_ref, ts_ref, dis_ref, b2_ref, o_ref):
    oc = o_ref.shape[1]
    v = (dis_ref[...] * (q0_ref[0, :, :oc] + q1_ref[0, :, :oc]
                         + ts_ref[:, :oc]) + b2_ref[...])
    m = jnp.max(v, axis=1, keepdims=True)
    e = jnp.exp(v - m)
    s = jnp.sum(e, axis=1, keepdims=True)
    o_ref[...] = v - m - jnp.log(s)


def _rows(br, c):
    return pl.BlockSpec((br, c), lambda i: (i, 0))


def _part(core, br, c):
    return pl.BlockSpec((1, br, c), lambda i, _core=core: (_core, i, 0))


def _full(shape):
    return pl.BlockSpec(shape, lambda i: tuple(0 for _ in shape))


def kernel(x, edge_index, W1, b1, W2, b2):
    n, in_ch = x.shape
    hid = W1.shape[1]
    out_ch = W2.shape[1]
    e = edge_index.shape[1]
    src = edge_index[0].astype(jnp.int32)
    dst = edge_index[1].astype(jnp.int32)

    align = NS * 128
    npad = ((n + align - 1) // align) * align

    br = 2000
    grid = (n // br,)

    degp = _make_deg_hist(npad, e)(dst).reshape(NC, npad, 1)

    dis, xs = pl.pallas_call(
        _scale_kernel,
        grid=grid,
        in_specs=[_part(0, br, 1), _part(1, br, 1), _rows(br, in_ch)],
        out_specs=[_rows(br, 1), _rows(br, in_ch)],
        out_shape=[jax.ShapeDtypeStruct((n, 1), jnp.float32),
                   jax.ShapeDtypeStruct((n, in_ch), jnp.float32)],
    )(degp, degp, x)

    p = _make_edge_agg(npad, e, in_ch)(xs, src, dst)

    ts = pl.pallas_call(
        _mlp_kernel,
        grid=grid,
        in_specs=[_part(0, br, in_ch), _part(1, br, in_ch), _rows(br, in_ch),
                  _rows(br, 1), _full((in_ch, hid)), _full((1, hid)),
                  _full((hid, out_ch))],
        out_specs=_rows(br, 128),
        out_shape=jax.ShapeDtypeStruct((n, 128), jnp.float32),
    )(p, p, xs, dis, W1, b1.reshape(1, hid), W2)

    q = _make_edge_agg(npad, e, 128)(ts, src, dst)

    out = pl.pallas_call(
        _out_kernel,
        grid=grid,
        in_specs=[_part(0, br, 128), _part(1, br, 128), _rows(br, 128),
                  _rows(br, 1), _full((1, out_ch))],
        out_specs=_rows(br, out_ch),
        out_shape=jax.ShapeDtypeStruct((n, out_ch), jnp.float32),
    )(q, q, ts, dis, b2.reshape(1, out_ch))

    return out


# speedup vs baseline: 1.0547x; 1.0011x over previous
import jax
import jax.numpy as jnp
from jax import lax
from jax.experimental import pallas as pl
from jax.experimental.pallas import tpu as pltpu
from jax.experimental.pallas import tpu_sc as plsc

NC = 2
NS = 16
LANES = 16


def _zero_fill(ref, rows, cols):
    zv = jnp.zeros((LANES,), jnp.float32)
    cpr = cols // LANES

    def body(i, c):
        r = i // cpr
        col = (i % cpr) * LANES
        ref[r, pl.ds(col, LANES)] = zv
        return c

    lax.fori_loop(0, rows * cpr, body, 0)


def _make_edge_agg(n_nodes, n_edges, n_ch):
    nw = NC * NS
    epw = n_edges // nw
    K = 80
    assert n_edges % nw == 0 and epw % K == 0
    nchunks = epw // K
    stripe = n_nodes // NS
    assert n_nodes % NS == 0 and stripe % 8 == 0
    zr = 128
    assert stripe % zr == 0

    mesh = plsc.VectorSubcoreMesh(core_axis_name="c", subcore_axis_name="s")

    nbuf = 3
    scratch = [
        [pltpu.VMEM((K,), jnp.int32) for _ in range(nbuf)],
        [pltpu.VMEM((K,), jnp.int32) for _ in range(nbuf)],
        [pltpu.VMEM((K, n_ch), jnp.float32) for _ in range(nbuf)],
        pltpu.VMEM((zr, n_ch), jnp.float32),
        pltpu.VMEM_SHARED((n_nodes, n_ch), jnp.float32),
        [pltpu.SemaphoreType.DMA for _ in range(nbuf)],
        [pltpu.SemaphoreType.DMA for _ in range(nbuf)],
    ]

    def body(table_hbm, src_hbm, dst_hbm, out_hbm,
             src_v, dst_v, rows_v, z_v, acc_sh, gsem, ssem):
        cid = lax.axis_index("c")
        sid = lax.axis_index("s")
        wid = sid * NC + cid

        _zero_fill(z_v, zr, n_ch)
        def zcopy(j, c):
            pltpu.sync_copy(z_v, acc_sh.at[pl.ds(sid * stripe + j * zr, zr)])
            return c
        lax.fori_loop(0, stripe // zr, zcopy, 0)

        plsc.subcore_barrier()

        base = wid * epw

        def start(g, b):
            off = pl.multiple_of(base + g * K, 8)
            pltpu.sync_copy(dst_hbm.at[pl.ds(off, K)], dst_v[b])
            pltpu.sync_copy(src_hbm.at[pl.ds(off, K)], src_v[b])
            pltpu.async_copy(table_hbm.at[src_v[b]], rows_v[b], gsem[b])

        def finish(b):
            pltpu.make_async_copy(table_hbm.at[src_v[b]], rows_v[b],
                                  gsem[b]).wait()
            pltpu.async_copy(rows_v[b], acc_sh.at[dst_v[b]], ssem[b],
                             add=True)

        def wait_scatter(b):
            pltpu.make_async_copy(rows_v[b], acc_sh.at[dst_v[b]],
                                  ssem[b]).wait()

        la = nbuf - 1
        for g in range(la):
            start(g, g)

        start(la, la % nbuf)
        finish(0)

        def outer(g0, c):
            for db in range(nbuf):
                g = g0 + db
                b = (1 + db) % nbuf

                @pl.when(g < nchunks)
                def _():
                    @pl.when(g + la < nchunks)
                    def _():
                        nb = (b + la) % nbuf
                        wait_scatter(nb)
                        start(g + la, nb)
                    finish(b)
            return c
        lax.fori_loop(0, (nchunks - 1 + nbuf - 1) // nbuf,
                      lambda i, c: outer(1 + i * nbuf, c), 0)

        for b in range(nbuf):
            wait_scatter(b)

        plsc.subcore_barrier()

        row0 = sid * stripe
        pltpu.sync_copy(acc_sh.at[pl.ds(row0, stripe)],
                        out_hbm.at[cid, pl.ds(row0, stripe)])

    return pl.kernel(
        body,
        out_type=jax.ShapeDtypeStruct((NC, n_nodes, n_ch), jnp.float32),
        mesh=mesh,
        scratch_types=scratch,
    )


def _make_deg_hist(n_nodes, n_edges):
    nw = NC * NS
    epw = n_edges // nw
    K = 80
    assert n_edges % nw == 0 and epw % K == 0
    nchunks = epw // K
    stripe = n_nodes // NS
    assert n_nodes % NS == 0 and stripe % 8 == 0

    mesh = plsc.VectorSubcoreMesh(core_axis_name="c", subcore_axis_name="s")

    nbuf = 3

    def body(dst_hbm, out_hbm, dst_v, ones_v, z_v, acc_sh, sem):
        cid = lax.axis_index("c")
        sid = lax.axis_index("s")
        wid = sid * NC + cid
        zv = jnp.zeros((LANES,), jnp.float32)
        ov = jnp.ones((LANES,), jnp.float32)

        def zf(i, c):
            z_v[pl.ds(i * LANES, LANES)] = zv
            return c
        lax.fori_loop(0, stripe // LANES, zf, 0)

        def of(i, c):
            ones_v[pl.ds(i * LANES, LANES)] = ov
            return c
        lax.fori_loop(0, K // LANES, of, 0)

        pltpu.sync_copy(z_v, acc_sh.at[pl.ds(sid * stripe, stripe)])
        plsc.subcore_barrier()

        base = wid * epw

        def start(g, b):
            off = pl.multiple_of(base + g * K, 8)
            pltpu.async_copy(dst_hbm.at[pl.ds(off, K)], dst_v[b], sem[b])

        def finish(g, b):
            off = pl.multiple_of(base + g * K, 8)
            pltpu.make_async_copy(dst_hbm.at[pl.ds(off, K)], dst_v[b],
                                  sem[b]).wait()
            pltpu.sync_copy(ones_v, acc_sh.at[dst_v[b]], add=True)

        la = nbuf - 1
        for g in range(la):
            start(g, g)

        def outer(g0, c):
            for b in range(nbuf):
                g = g0 + b

                @pl.when(g < nchunks)
                def _():
                    @pl.when(g + la < nchunks)
                    def _():
                        start(g + la, (b + la) % nbuf)
                    finish(g, b)
            return c
        lax.fori_loop(0, (nchunks + nbuf - 1) // nbuf,
                      lambda i, c: outer(i * nbuf, c), 0)

        plsc.subcore_barrier()
        pltpu.sync_copy(acc_sh.at[pl.ds(sid * stripe, stripe)],
                        out_hbm.at[cid, pl.ds(sid * stripe, stripe)])

    return pl.kernel(
        body,
        out_type=jax.ShapeDtypeStruct((NC, n_nodes), jnp.float32),
        mesh=mesh,
        scratch_types=[
            [pltpu.VMEM((K,), jnp.int32) for _ in range(nbuf)],
            pltpu.VMEM((K,), jnp.float32),
            pltpu.VMEM((stripe,), jnp.float32),
            pltpu.VMEM_SHARED((n_nodes,), jnp.float32),
            [pltpu.SemaphoreType.DMA for _ in range(nbuf)],
        ],
    )


def _scale_kernel(d0_ref, d1_ref, x_ref, dis_ref, xs_ref):
    deg = d0_ref[0] + d1_ref[0] + 1.0
    dis = lax.rsqrt(deg)
    dis_ref[...] = dis
    xs_ref[...] = x_ref[...] * dis


def _mlp_kernel(p0_ref, p1_ref, xs_ref, dis_ref, w1_ref, b1_ref, w2_ref,
                ts_ref):
    dis = dis_ref[...]
    a = dis * (p0_ref[0] + p1_ref[0] + xs_ref[...])
    h = jnp.maximum(
        jnp.dot(a, w1_ref[...], preferred_element_type=jnp.float32)
        + b1_ref[...], 0.0)
    t = jnp.dot(h, w2_ref[...], preferred_element_type=jnp.float32)
    br, oc = t.shape
    ts_ref[...] = jnp.concatenate(
        [dis * t, jnp.zeros((br, 128 - oc), jnp.float32)], axis=1)


def _out_kernel(q0_ref, q1_ref, ts_ref, dis_ref, b2_ref, o_ref):
    oc = o_ref.shape[1]
    v = (dis_ref[...] * (q0_ref[0, :, :oc] + q1_ref[0, :, :oc]
                         + ts_ref[:, :oc]) + b2_ref[...])
    m = jnp.max(v, axis=1, keepdims=True)
    e = jnp.exp(v - m)
    s = jnp.sum(e, axis=1, keepdims=True)
    o_ref[...] = v - m - jnp.log(s)


def _rows(br, c):
    return pl.BlockSpec((br, c), lambda i: (i, 0))


def _part(core, br, c):
    return pl.BlockSpec((1, br, c), lambda i, _core=core: (_core, i, 0))


def _full(shape):
    return pl.BlockSpec(shape, lambda i: tuple(0 for _ in shape))


def kernel(x, edge_index, W1, b1, W2, b2):
    n, in_ch = x.shape
    hid = W1.shape[1]
    out_ch = W2.shape[1]
    e = edge_index.shape[1]
    src = edge_index[0].astype(jnp.int32)
    dst = edge_index[1].astype(jnp.int32)

    align = NS * 128
    npad = ((n + align - 1) // align) * align

    br = 2000
    grid = (n // br,)

    degp = _make_deg_hist(npad, e)(dst).reshape(NC, npad, 1)

    dis, xs = pl.pallas_call(
        _scale_kernel,
        grid=grid,
        in_specs=[_part(0, br, 1), _part(1, br, 1), _rows(br, in_ch)],
        out_specs=[_rows(br, 1), _rows(br, in_ch)],
        out_shape=[jax.ShapeDtypeStruct((n, 1), jnp.float32),
                   jax.ShapeDtypeStruct((n, in_ch), jnp.float32)],
    )(degp, degp, x)

    p = _make_edge_agg(npad, e, in_ch)(xs, src, dst)

    ts = pl.pallas_call(
        _mlp_kernel,
        grid=grid,
        in_specs=[_part(0, br, in_ch), _part(1, br, in_ch), _rows(br, in_ch),
                  _rows(br, 1), _full((in_ch, hid)), _full((1, hid)),
                  _full((hid, out_ch))],
        out_specs=_rows(br, 128),
        out_shape=jax.ShapeDtypeStruct((n, 128), jnp.float32),
    )(p, p, xs, dis, W1, b1.reshape(1, hid), W2)

    q = _make_edge_agg(npad, e, 128)(ts, src, dst)

    out = pl.pallas_call(
        _out_kernel,
        grid=grid,
        in_specs=[_part(0, br, 128), _part(1, br, 128), _rows(br, 128),
                  _rows(br, 1), _full((1, out_ch))],
        out_specs=_rows(br, out_ch),
        out_shape=jax.ShapeDtypeStruct((n, out_ch), jnp.float32),
    )(q, q, ts, dis, b2.reshape(1, out_ch))

    return out
